# baseline (speedup 1.0000x reference)
import functools

import jax
import jax.numpy as jnp
from jax import lax
from jax.experimental import pallas as pl
from jax.experimental.pallas import tpu as pltpu
from jax.experimental.pallas import tpu_sc as plsc

B = 16384
H = 50
D = 128
K = 128
LANES = 16
DV = D // LANES

NC = 2
NS = 16
NW = NC * NS

RPG = 2
IPG = RPG * H
G_PER_W = B // (RPG * NW)
NBUF = 8
FLUSH_G = 32
FLUSH_ROWS = FLUSH_G * RPG
NT = H // LANES
TL = H - NT * LANES


def _pool_body(ids_hbm, mask_hbm, table_hbm, out_hbm,
               ids_v, mask_v, rows_v, out_v, *sems):
    c = lax.axis_index("c")
    s = lax.axis_index("s")
    w = s * NC + c
    g0 = pl.multiple_of(w * G_PER_W, G_PER_W)

    pltpu.sync_copy(ids_hbm.at[pl.ds(g0, G_PER_W)], ids_v)
    pltpu.sync_copy(mask_hbm.at[pl.ds(g0, G_PER_W)], mask_v)

    def fire(g, slot):
        pltpu.make_async_copy(
            table_hbm.at[ids_v.at[g]], rows_v.at[slot], sems[slot]).start()

    def drain(g, slot):
        pltpu.make_async_copy(
            table_hbm.at[ids_v.at[g]], rows_v.at[slot], sems[slot]).wait()

    for slot in range(NBUF):
        fire(jnp.int32(slot), slot)

    zeros = jnp.zeros((LANES,), jnp.float32)
    iota = lax.iota(jnp.int32, LANES)

    def accum2(acc, mvec, k0, slot, j0):
        m0 = jnp.full((LANES,), mvec[k0], jnp.float32)
        m1 = jnp.full((LANES,), mvec[k0 + 1], jnp.float32)
        mb0 = plsc.pack(m0, m0, format=plsc.PackFormat.INTERLEAVED)
        mb1 = plsc.pack(m1, m1, format=plsc.PackFormat.INTERLEAVED)
        out = list(acc)
        for q in range(DV // 2):
            rv0 = rows_v[slot, j0, pl.ds(q * 2 * LANES, 2 * LANES)]
            rv1 = rows_v[slot, j0 + 1, pl.ds(q * 2 * LANES, 2 * LANES)]
            pa, pb = plsc.unpack(rv0 * mb0 + rv1 * mb1,
                                 format=plsc.PackFormat.INTERLEAVED)
            out[2 * q] = acc[2 * q] + pa
            out[2 * q + 1] = acc[2 * q + 1] + pb
        return out

    def gg_body(gg, carry):
        for slot in range(NBUF):
            g = gg * NBUF + slot
            drain(g, slot)
            for r in range(RPG):
                base = r * H

                def t_body(t, tc):
                    off = base + t * LANES
                    mvec = mask_v[g, pl.ds(off, LANES)]
                    acc = list(tc[:DV])
                    for k in range(0, LANES, 2):
                        acc = accum2(acc, mvec, k, slot, off + k)
                    return (*acc, tc[DV] + mvec)

                res = lax.fori_loop(0, NT, t_body, (zeros,) * (DV + 1))
                acc, wsv = list(res[:DV]), res[DV]
                mvec_t = mask_v[g, pl.ds(base + H - LANES, LANES)]
                wtot = (jnp.full((LANES,), jnp.sum(wsv), jnp.float32)
                        + jnp.full((LANES,), mvec_t[LANES - TL], jnp.float32)
                        + jnp.full((LANES,), mvec_t[LANES - 1], jnp.float32))
                acc = accum2(acc, mvec_t, LANES - TL, slot,
                             base + NT * LANES)
                recip = 1.0 / wtot
                lr = (g % FLUSH_G) * RPG + r
                ibase = lr * D
                for q in range(DV // 2):
                    idx = ibase + q * 2 * LANES + 2 * iota
                    plsc.store_scatter(out_v, [idx], acc[2 * q] * recip)
                    plsc.store_scatter(out_v, [idx + 1],
                                       acc[2 * q + 1] * recip)

            @pl.when(g + NBUF < G_PER_W)
            def _():
                fire(g + NBUF, slot)

            @pl.when((g + 1) % FLUSH_G == 0)
            def _():
                obase = pl.multiple_of(
                    (w * (G_PER_W * RPG) + (g + 1 - FLUSH_G) * RPG) * D,
                    FLUSH_ROWS * D)
                pltpu.sync_copy(out_v,
                                out_hbm.at[pl.ds(obase, FLUSH_ROWS * D)])
        return carry

    lax.fori_loop(0, G_PER_W // NBUF, gg_body, jnp.int32(0))


_pool_kernel = functools.partial(
    pl.kernel,
    out_type=jax.ShapeDtypeStruct((B * D,), jnp.float32),
    mesh=plsc.VectorSubcoreMesh(core_axis_name="c", subcore_axis_name="s"),
    scratch_types=[
        pltpu.VMEM((G_PER_W, IPG), jnp.int32),
        pltpu.VMEM((G_PER_W, IPG), jnp.float32),
        pltpu.VMEM((NBUF, IPG, D), jnp.bfloat16),
        pltpu.VMEM((FLUSH_ROWS * D,), jnp.float32),
    ] + [pltpu.SemaphoreType.DMA] * NBUF,
    compiler_params=pltpu.CompilerParams(
        needs_layout_passes=False, use_tc_tiling_on_sc=False),
)(_pool_body)


def _cast_body(x_ref, o_ref):
    o_ref[...] = x_ref[...].astype(jnp.bfloat16)


def _cast_bf16(table):
    v, blk = table.shape[0], 2000
    return pl.pallas_call(
        _cast_body,
        grid=(v // blk,),
        in_specs=[pl.BlockSpec((blk, D), lambda i: (i, 0))],
        out_specs=pl.BlockSpec((blk, D), lambda i: (i, 0)),
        out_shape=jax.ShapeDtypeStruct((v, D), jnp.bfloat16),
    )(table)


def _dense_body(x_ref, w_ref, b_ref, g_ref, be_ref, o_ref):
    y = lax.dot_general(x_ref[...], w_ref[...], (((1,), (1,)), ((), ())),
                        preferred_element_type=jnp.float32)
    y = y + b_ref[...]
    mu = jnp.mean(y, axis=-1, keepdims=True)
    yc = y - mu
    var = jnp.mean(yc * yc, axis=-1, keepdims=True)
    y = yc * lax.rsqrt(var + 1e-5) * g_ref[...] + be_ref[...]
    o_ref[...] = jnp.maximum(y, 0.0)


def _dense(x, w, bvec, gamma, beta):
    blk = 2048
    return pl.pallas_call(
        _dense_body,
        grid=(B // blk,),
        in_specs=[
            pl.BlockSpec((blk, D), lambda i: (i, 0)),
            pl.BlockSpec((K, D), lambda i: (0, 0)),
            pl.BlockSpec((1, K), lambda i: (0, 0)),
            pl.BlockSpec((1, K), lambda i: (0, 0)),
            pl.BlockSpec((1, K), lambda i: (0, 0)),
        ],
        out_specs=pl.BlockSpec((blk, K), lambda i: (i, 0)),
        out_shape=jax.ShapeDtypeStruct((B, K), jnp.float32),
    )(x, w, bvec, gamma, beta)


def kernel(keyword_ids, keyword_mask, table, W, b, gamma, beta):
    ids2 = keyword_ids.reshape(B // RPG, IPG).astype(jnp.int32)
    mask2 = keyword_mask.reshape(B // RPG, IPG)
    pooled = _pool_kernel(ids2, mask2, table.astype(jnp.bfloat16))
    return _dense(pooled.reshape(B, D), W, b.reshape(1, K),
                  gamma.reshape(1, K), beta.reshape(1, K))


# speedup vs baseline: 1.1791x; 1.1791x over previous
import functools

import jax
import jax.numpy as jnp
from jax import lax
from jax.experimental import pallas as pl
from jax.experimental.pallas import tpu as pltpu
from jax.experimental.pallas import tpu_sc as plsc

B = 16384
H = 50
D = 128
K = 128
LANES = 16
DV = D // LANES

NC = 2
NS = 16
NW = NC * NS

RPG = 2
IPG = RPG * H
G_PER_W = B // (RPG * NW)
NBUF = 4
FLUSH_G = 32
FLUSH_ROWS = FLUSH_G * RPG
NT = H // LANES
TL = H - NT * LANES


def _pool_body(ids_hbm, mask_hbm, table_hbm, out_hbm,
               ids_v, mask_v, rows_v, out_v, *sems):
    c = lax.axis_index("c")
    s = lax.axis_index("s")
    w = s * NC + c
    g0 = pl.multiple_of(w * G_PER_W, G_PER_W)

    pltpu.sync_copy(ids_hbm.at[pl.ds(g0, G_PER_W)], ids_v)
    pltpu.sync_copy(mask_hbm.at[pl.ds(g0, G_PER_W)], mask_v)

    def fire(g, slot):
        pltpu.make_async_copy(
            table_hbm.at[ids_v.at[g]], rows_v.at[slot], sems[slot]).start()

    def drain(g, slot):
        pltpu.make_async_copy(
            table_hbm.at[ids_v.at[g]], rows_v.at[slot], sems[slot]).wait()

    for slot in range(NBUF):
        fire(jnp.int32(slot), slot)

    zeros = jnp.zeros((LANES,), jnp.float32)
    iota = lax.iota(jnp.int32, LANES)

    def accum2(acc, mvec, k0, slot, j0):
        m0 = jnp.full((LANES,), mvec[k0], jnp.float32)
        m1 = jnp.full((LANES,), mvec[k0 + 1], jnp.float32)
        mb0 = plsc.pack(m0, m0, format=plsc.PackFormat.INTERLEAVED)
        mb1 = plsc.pack(m1, m1, format=plsc.PackFormat.INTERLEAVED)
        out = list(acc)
        for q in range(DV // 2):
            rv0 = rows_v[slot, j0, pl.ds(q * 2 * LANES, 2 * LANES)]
            rv1 = rows_v[slot, j0 + 1, pl.ds(q * 2 * LANES, 2 * LANES)]
            pa, pb = plsc.unpack(rv0 * mb0 + rv1 * mb1,
                                 format=plsc.PackFormat.INTERLEAVED)
            out[2 * q] = acc[2 * q] + pa
            out[2 * q + 1] = acc[2 * q + 1] + pb
        return out

    def accum4(acc, mvec, k0, slot, j0):
        mb = []
        for k in range(4):
            m = jnp.full((LANES,), mvec[k0 + k], jnp.float32)
            mb.append(plsc.pack(m, m, format=plsc.PackFormat.INTERLEAVED))
        out = list(acc)
        for q in range(DV // 2):
            sl = pl.ds(q * 2 * LANES, 2 * LANES)
            s01 = rows_v[slot, j0, sl] * mb[0] + rows_v[slot, j0 + 1, sl] * mb[1]
            s23 = rows_v[slot, j0 + 2, sl] * mb[2] + rows_v[slot, j0 + 3, sl] * mb[3]
            pa, pb = plsc.unpack(s01 + s23,
                                 format=plsc.PackFormat.INTERLEAVED)
            out[2 * q] = acc[2 * q] + pa
            out[2 * q + 1] = acc[2 * q + 1] + pb
        return out

    def gg_body(gg, carry):
        for slot in range(NBUF):
            g = gg * NBUF + slot
            drain(g, slot)
            for r in range(RPG):
                base = r * H

                def t_body(t, tc):
                    off = base + t * LANES
                    mvec = mask_v[g, pl.ds(off, LANES)]
                    acc = list(tc[:DV])
                    for k in range(0, LANES, 4):
                        acc = accum4(acc, mvec, k, slot, off + k)
                    return (*acc, tc[DV] + mvec)

                res = lax.fori_loop(0, NT, t_body, (zeros,) * (DV + 1))
                acc, wsv = list(res[:DV]), res[DV]
                mvec_t = mask_v[g, pl.ds(base + H - LANES, LANES)]
                wtot = (jnp.full((LANES,), jnp.sum(wsv), jnp.float32)
                        + jnp.full((LANES,), mvec_t[LANES - TL], jnp.float32)
                        + jnp.full((LANES,), mvec_t[LANES - 1], jnp.float32))
                acc = accum2(acc, mvec_t, LANES - TL, slot,
                             base + NT * LANES)
                recip = 1.0 / wtot
                lr = (g % FLUSH_G) * RPG + r
                ibase = lr * D
                for q in range(DV // 2):
                    idx = ibase + q * 2 * LANES + 2 * iota
                    plsc.store_scatter(out_v, [idx], acc[2 * q] * recip)
                    plsc.store_scatter(out_v, [idx + 1],
                                       acc[2 * q + 1] * recip)

            @pl.when(g + NBUF < G_PER_W)
            def _():
                fire(g + NBUF, slot)

            @pl.when((g + 1) % FLUSH_G == 0)
            def _():
                obase = pl.multiple_of(
                    (w * (G_PER_W * RPG) + (g + 1 - FLUSH_G) * RPG) * D,
                    FLUSH_ROWS * D)
                pltpu.sync_copy(out_v,
                                out_hbm.at[pl.ds(obase, FLUSH_ROWS * D)])
        return carry

    lax.fori_loop(0, G_PER_W // NBUF, gg_body, jnp.int32(0))


_pool_kernel = functools.partial(
    pl.kernel,
    out_type=jax.ShapeDtypeStruct((B * D,), jnp.float32),
    mesh=plsc.VectorSubcoreMesh(core_axis_name="c", subcore_axis_name="s"),
    scratch_types=[
        pltpu.VMEM((G_PER_W, IPG), jnp.int32),
        pltpu.VMEM((G_PER_W, IPG), jnp.float32),
        pltpu.VMEM((NBUF, IPG, D), jnp.bfloat16),
        pltpu.VMEM((FLUSH_ROWS * D,), jnp.float32),
    ] + [pltpu.SemaphoreType.DMA] * NBUF,
    compiler_params=pltpu.CompilerParams(
        needs_layout_passes=False, use_tc_tiling_on_sc=False),
)(_pool_body)


def _cast_body(x_ref, o_ref):
    o_ref[...] = x_ref[...].astype(jnp.bfloat16)


def _cast_bf16(table):
    v, blk = table.shape[0], 2000
    return pl.pallas_call(
        _cast_body,
        grid=(v // blk,),
        in_specs=[pl.BlockSpec((blk, D), lambda i: (i, 0))],
        out_specs=pl.BlockSpec((blk, D), lambda i: (i, 0)),
        out_shape=jax.ShapeDtypeStruct((v, D), jnp.bfloat16),
    )(table)


def _dense_body(x_ref, w_ref, b_ref, g_ref, be_ref, o_ref):
    y = lax.dot_general(x_ref[...], w_ref[...], (((1,), (1,)), ((), ())),
                        preferred_element_type=jnp.float32)
    y = y + b_ref[...]
    mu = jnp.mean(y, axis=-1, keepdims=True)
    yc = y - mu
    var = jnp.mean(yc * yc, axis=-1, keepdims=True)
    y = yc * lax.rsqrt(var + 1e-5) * g_ref[...] + be_ref[...]
    o_ref[...] = jnp.maximum(y, 0.0)


def _dense(x, w, bvec, gamma, beta):
    blk = 2048
    return pl.pallas_call(
        _dense_body,
        grid=(B // blk,),
        in_specs=[
            pl.BlockSpec((blk, D), lambda i: (i, 0)),
            pl.BlockSpec((K, D), lambda i: (0, 0)),
            pl.BlockSpec((1, K), lambda i: (0, 0)),
            pl.BlockSpec((1, K), lambda i: (0, 0)),
            pl.BlockSpec((1, K), lambda i: (0, 0)),
        ],
        out_specs=pl.BlockSpec((blk, K), lambda i: (i, 0)),
        out_shape=jax.ShapeDtypeStruct((B, K), jnp.float32),
    )(x, w, bvec, gamma, beta)


def kernel(keyword_ids, keyword_mask, table, W, b, gamma, beta):
    ids2 = keyword_ids.reshape(B // RPG, IPG).astype(jnp.int32)
    mask2 = keyword_mask.reshape(B // RPG, IPG)
    pooled = _pool_kernel(ids2, mask2, table.astype(jnp.bfloat16))
    return _dense(pooled.reshape(B, D), W, b.reshape(1, K),
                  gamma.reshape(1, K), beta.reshape(1, K))


# speedup vs baseline: 1.2103x; 1.0264x over previous
import functools

import jax
import jax.numpy as jnp
from jax import lax
from jax.experimental import pallas as pl
from jax.experimental.pallas import tpu as pltpu
from jax.experimental.pallas import tpu_sc as plsc

B = 16384
H = 50
D = 128
K = 128
LANES = 16
DV = D // LANES

NC = 2
NS = 16
NW = NC * NS

RPG = 2
IPG = RPG * H
G_PER_W = B // (RPG * NW)
NBUF = 4
FLUSH_G = 32
FLUSH_ROWS = FLUSH_G * RPG
NT = H // LANES
TL = H - NT * LANES


def _pool_body(ids_hbm, mask_hbm, table_hbm, out_hbm,
               ids_v, mask_v, rows_v, out_v, *sems):
    c = lax.axis_index("c")
    s = lax.axis_index("s")
    w = s * NC + c
    g0 = pl.multiple_of(w * G_PER_W, G_PER_W)

    pltpu.sync_copy(ids_hbm.at[pl.ds(g0, G_PER_W)], ids_v)
    pltpu.sync_copy(mask_hbm.at[pl.ds(g0, G_PER_W)], mask_v)

    def fire(g, slot):
        pltpu.make_async_copy(
            table_hbm.at[ids_v.at[g]], rows_v.at[slot], sems[slot]).start()

    def drain(g, slot):
        pltpu.make_async_copy(
            table_hbm.at[ids_v.at[g]], rows_v.at[slot], sems[slot]).wait()

    for slot in range(NBUF):
        fire(jnp.int32(slot), slot)

    zeros = jnp.zeros((LANES,), jnp.float32)
    iota = lax.iota(jnp.int32, LANES)

    def accum2(acc, mvec, k0, slot, j0):
        m0 = jnp.full((LANES,), mvec[k0], jnp.float32)
        m1 = jnp.full((LANES,), mvec[k0 + 1], jnp.float32)
        mb0 = plsc.pack(m0, m0, format=plsc.PackFormat.INTERLEAVED)
        mb1 = plsc.pack(m1, m1, format=plsc.PackFormat.INTERLEAVED)
        out = list(acc)
        for q in range(DV // 2):
            rv0 = rows_v[slot, j0, pl.ds(q * 2 * LANES, 2 * LANES)]
            rv1 = rows_v[slot, j0 + 1, pl.ds(q * 2 * LANES, 2 * LANES)]
            pa, pb = plsc.unpack(rv0 * mb0 + rv1 * mb1,
                                 format=plsc.PackFormat.INTERLEAVED)
            out[2 * q] = acc[2 * q] + pa
            out[2 * q + 1] = acc[2 * q + 1] + pb
        return out

    def accum4(acc, mvec, k0, slot, j0):
        mb = []
        for k in range(4):
            m = jnp.full((LANES,), mvec[k0 + k], jnp.float32)
            mb.append(plsc.pack(m, m, format=plsc.PackFormat.INTERLEAVED))
        out = list(acc)
        for q in range(DV // 2):
            sl = pl.ds(q * 2 * LANES, 2 * LANES)
            s01 = rows_v[slot, j0, sl] * mb[0] + rows_v[slot, j0 + 1, sl] * mb[1]
            s23 = rows_v[slot, j0 + 2, sl] * mb[2] + rows_v[slot, j0 + 3, sl] * mb[3]
            pa, pb = plsc.unpack(s01 + s23,
                                 format=plsc.PackFormat.INTERLEAVED)
            out[2 * q] = acc[2 * q] + pa
            out[2 * q + 1] = acc[2 * q + 1] + pb
        return out

    def accum8(acc, mvec, k0, slot, j0):
        mb = []
        for k in range(8):
            m = jnp.full((LANES,), mvec[k0 + k], jnp.float32)
            mb.append(plsc.pack(m, m, format=plsc.PackFormat.INTERLEAVED))
        out = list(acc)
        for q in range(DV // 2):
            sl = pl.ds(q * 2 * LANES, 2 * LANES)
            p = [rows_v[slot, j0 + k, sl] * mb[k] for k in range(8)]
            s = ((p[0] + p[1]) + (p[2] + p[3])) + ((p[4] + p[5]) + (p[6] + p[7]))
            pa, pb = plsc.unpack(s, format=plsc.PackFormat.INTERLEAVED)
            out[2 * q] = acc[2 * q] + pa
            out[2 * q + 1] = acc[2 * q + 1] + pb
        return out

    def gg_body(gg, carry):
        for slot in range(NBUF):
            g = gg * NBUF + slot
            drain(g, slot)
            for r in range(RPG):
                base = r * H

                def t_body(t, tc):
                    off = base + t * LANES
                    mvec = mask_v[g, pl.ds(off, LANES)]
                    acc = list(tc[:DV])
                    for k in range(0, LANES, 8):
                        acc = accum8(acc, mvec, k, slot, off + k)
                    return (*acc, tc[DV] + mvec)

                res = lax.fori_loop(0, NT, t_body, (zeros,) * (DV + 1))
                acc, wsv = list(res[:DV]), res[DV]
                mvec_t = mask_v[g, pl.ds(base + H - LANES, LANES)]
                wtot = (jnp.full((LANES,), jnp.sum(wsv), jnp.float32)
                        + jnp.full((LANES,), mvec_t[LANES - TL], jnp.float32)
                        + jnp.full((LANES,), mvec_t[LANES - 1], jnp.float32))
                acc = accum2(acc, mvec_t, LANES - TL, slot,
                             base + NT * LANES)
                recip = 1.0 / wtot
                lr = (g % FLUSH_G) * RPG + r
                ibase = lr * D
                for q in range(DV // 2):
                    idx = ibase + q * 2 * LANES + 2 * iota
                    plsc.store_scatter(out_v, [idx], acc[2 * q] * recip)
                    plsc.store_scatter(out_v, [idx + 1],
                                       acc[2 * q + 1] * recip)

            @pl.when(g + NBUF < G_PER_W)
            def _():
                fire(g + NBUF, slot)

            @pl.when((g + 1) % FLUSH_G == 0)
            def _():
                obase = pl.multiple_of(
                    (w * (G_PER_W * RPG) + (g + 1 - FLUSH_G) * RPG) * D,
                    FLUSH_ROWS * D)
                pltpu.sync_copy(out_v,
                                out_hbm.at[pl.ds(obase, FLUSH_ROWS * D)])
        return carry

    lax.fori_loop(0, G_PER_W // NBUF, gg_body, jnp.int32(0))


_pool_kernel = functools.partial(
    pl.kernel,
    out_type=jax.ShapeDtypeStruct((B * D,), jnp.float32),
    mesh=plsc.VectorSubcoreMesh(core_axis_name="c", subcore_axis_name="s"),
    scratch_types=[
        pltpu.VMEM((G_PER_W, IPG), jnp.int32),
        pltpu.VMEM((G_PER_W, IPG), jnp.float32),
        pltpu.VMEM((NBUF, IPG, D), jnp.bfloat16),
        pltpu.VMEM((FLUSH_ROWS * D,), jnp.float32),
    ] + [pltpu.SemaphoreType.DMA] * NBUF,
    compiler_params=pltpu.CompilerParams(
        needs_layout_passes=False, use_tc_tiling_on_sc=False),
)(_pool_body)


def _cast_body(x_ref, o_ref):
    o_ref[...] = x_ref[...].astype(jnp.bfloat16)


def _cast_bf16(table):
    v, blk = table.shape[0], 2000
    return pl.pallas_call(
        _cast_body,
        grid=(v // blk,),
        in_specs=[pl.BlockSpec((blk, D), lambda i: (i, 0))],
        out_specs=pl.BlockSpec((blk, D), lambda i: (i, 0)),
        out_shape=jax.ShapeDtypeStruct((v, D), jnp.bfloat16),
    )(table)


def _dense_body(x_ref, w_ref, b_ref, g_ref, be_ref, o_ref):
    y = lax.dot_general(x_ref[...], w_ref[...], (((1,), (1,)), ((), ())),
                        preferred_element_type=jnp.float32)
    y = y + b_ref[...]
    mu = jnp.mean(y, axis=-1, keepdims=True)
    yc = y - mu
    var = jnp.mean(yc * yc, axis=-1, keepdims=True)
    y = yc * lax.rsqrt(var + 1e-5) * g_ref[...] + be_ref[...]
    o_ref[...] = jnp.maximum(y, 0.0)


def _dense(x, w, bvec, gamma, beta):
    blk = 2048
    return pl.pallas_call(
        _dense_body,
        grid=(B // blk,),
        in_specs=[
            pl.BlockSpec((blk, D), lambda i: (i, 0)),
            pl.BlockSpec((K, D), lambda i: (0, 0)),
            pl.BlockSpec((1, K), lambda i: (0, 0)),
            pl.BlockSpec((1, K), lambda i: (0, 0)),
            pl.BlockSpec((1, K), lambda i: (0, 0)),
        ],
        out_specs=pl.BlockSpec((blk, K), lambda i: (i, 0)),
        out_shape=jax.ShapeDtypeStruct((B, K), jnp.float32),
    )(x, w, bvec, gamma, beta)


def kernel(keyword_ids, keyword_mask, table, W, b, gamma, beta):
    ids2 = keyword_ids.reshape(B // RPG, IPG).astype(jnp.int32)
    mask2 = keyword_mask.reshape(B // RPG, IPG)
    pooled = _pool_kernel(ids2, mask2, table.astype(jnp.bfloat16))
    return _dense(pooled.reshape(B, D), W, b.reshape(1, K),
                  gamma.reshape(1, K), beta.reshape(1, K))


# speedup vs baseline: 1.2168x; 1.0054x over previous
import functools

import jax
import jax.numpy as jnp
from jax import lax
from jax.experimental import pallas as pl
from jax.experimental.pallas import tpu as pltpu
from jax.experimental.pallas import tpu_sc as plsc

B = 16384
H = 50
D = 128
K = 128
LANES = 16
DV = D // LANES

NC = 2
NS = 16
NW = NC * NS

RPG = 2
IPG = RPG * H
G_PER_W = B // (RPG * NW)
NBUF = 4
FLUSH_G = 32
FLUSH_ROWS = FLUSH_G * RPG
NT = H // LANES
TL = H - NT * LANES


def _pool_body(ids_hbm, mask_hbm, table_hbm, out_hbm,
               ids_v, mask_v, rows_v, out_v, *sems):
    c = lax.axis_index("c")
    s = lax.axis_index("s")
    w = s * NC + c
    g0 = pl.multiple_of(w * G_PER_W, G_PER_W)

    pltpu.sync_copy(ids_hbm.at[pl.ds(g0, G_PER_W)], ids_v)
    pltpu.sync_copy(mask_hbm.at[pl.ds(g0, G_PER_W)], mask_v)

    def fire(g, slot):
        pltpu.make_async_copy(
            table_hbm.at[ids_v.at[g]], rows_v.at[slot], sems[slot]).start()

    def drain(g, slot):
        pltpu.make_async_copy(
            table_hbm.at[ids_v.at[g]], rows_v.at[slot], sems[slot]).wait()

    for slot in range(NBUF):
        fire(jnp.int32(slot), slot)

    zeros = jnp.zeros((LANES,), jnp.float32)
    iota = lax.iota(jnp.int32, LANES)

    def accum2(acc, mvec, k0, slot, j0):
        m0 = jnp.full((LANES,), mvec[k0], jnp.float32)
        m1 = jnp.full((LANES,), mvec[k0 + 1], jnp.float32)
        mb0 = plsc.pack(m0, m0, format=plsc.PackFormat.INTERLEAVED)
        mb1 = plsc.pack(m1, m1, format=plsc.PackFormat.INTERLEAVED)
        out = list(acc)
        for q in range(DV // 2):
            rv0 = rows_v[slot, j0, pl.ds(q * 2 * LANES, 2 * LANES)]
            rv1 = rows_v[slot, j0 + 1, pl.ds(q * 2 * LANES, 2 * LANES)]
            pa, pb = plsc.unpack(rv0 * mb0 + rv1 * mb1,
                                 format=plsc.PackFormat.INTERLEAVED)
            out[2 * q] = acc[2 * q] + pa
            out[2 * q + 1] = acc[2 * q + 1] + pb
        return out

    def accum8(acc, mvec, k0, slot, j0):
        mb = []
        for k in range(8):
            m = jnp.full((LANES,), mvec[k0 + k], jnp.float32)
            mb.append(plsc.pack(m, m, format=plsc.PackFormat.INTERLEAVED))
        out = list(acc)
        for q in range(DV // 2):
            sl = pl.ds(q * 2 * LANES, 2 * LANES)
            p = [rows_v[slot, j0 + k, sl] * mb[k] for k in range(8)]
            s = ((p[0] + p[1]) + (p[2] + p[3])) + ((p[4] + p[5]) + (p[6] + p[7]))
            pa, pb = plsc.unpack(s, format=plsc.PackFormat.INTERLEAVED)
            out[2 * q] = acc[2 * q] + pa
            out[2 * q + 1] = acc[2 * q + 1] + pb
        return out

    def gg_body(gg, carry):
        for slot in range(NBUF):
            g = gg * NBUF + slot
            drain(g, slot)
            for r in range(RPG):
                base = r * H

                def t_body(t, tc):
                    off = base + t * LANES
                    mvec = mask_v[g, pl.ds(off, LANES)]
                    acc = list(tc[:DV])
                    for k in range(0, LANES, 8):
                        acc = accum8(acc, mvec, k, slot, off + k)
                    return (*acc, tc[DV] + mvec)

                res = lax.fori_loop(0, NT, t_body, (zeros,) * (DV + 1))
                acc, wsv = list(res[:DV]), res[DV]
                mvec_t = mask_v[g, pl.ds(base + H - LANES, LANES)]
                wtot = (jnp.full((LANES,), jnp.sum(wsv), jnp.float32)
                        + jnp.full((LANES,), mvec_t[LANES - TL], jnp.float32)
                        + jnp.full((LANES,), mvec_t[LANES - 1], jnp.float32))
                acc = accum2(acc, mvec_t, LANES - TL, slot,
                             base + NT * LANES)
                recip = 1.0 / wtot
                lr = (g % FLUSH_G) * RPG + r
                ibase = lr * D
                for q in range(DV // 2):
                    idx = ibase + q * 2 * LANES + 2 * iota
                    plsc.store_scatter(out_v, [idx], acc[2 * q] * recip)
                    plsc.store_scatter(out_v, [idx + 1],
                                       acc[2 * q + 1] * recip)

            @pl.when(g + NBUF < G_PER_W)
            def _():
                fire(g + NBUF, slot)

            @pl.when((g + 1) % FLUSH_G == 0)
            def _():
                obase = pl.multiple_of(
                    (w * (G_PER_W * RPG) + (g + 1 - FLUSH_G) * RPG) * D,
                    FLUSH_ROWS * D)
                pltpu.sync_copy(out_v,
                                out_hbm.at[pl.ds(obase, FLUSH_ROWS * D)])
        return carry

    lax.fori_loop(0, G_PER_W // NBUF, gg_body, jnp.int32(0))


_pool_kernel = functools.partial(
    pl.kernel,
    out_type=jax.ShapeDtypeStruct((B * D,), jnp.float32),
    mesh=plsc.VectorSubcoreMesh(core_axis_name="c", subcore_axis_name="s"),
    scratch_types=[
        pltpu.VMEM((G_PER_W, IPG), jnp.int32),
        pltpu.VMEM((G_PER_W, IPG), jnp.float32),
        pltpu.VMEM((NBUF, IPG, D), jnp.bfloat16),
        pltpu.VMEM((FLUSH_ROWS * D,), jnp.float32),
    ] + [pltpu.SemaphoreType.DMA] * NBUF,
    compiler_params=pltpu.CompilerParams(
        needs_layout_passes=False, use_tc_tiling_on_sc=False),
)(_pool_body)


def _dense_body(x_ref, w_ref, b_ref, g_ref, be_ref, o_ref):
    y = lax.dot_general(x_ref[...], w_ref[...], (((1,), (1,)), ((), ())),
                        preferred_element_type=jnp.float32)
    y = y + b_ref[...]
    mu = jnp.mean(y, axis=-1, keepdims=True)
    yc = y - mu
    var = jnp.mean(yc * yc, axis=-1, keepdims=True)
    y = yc * lax.rsqrt(var + 1e-5) * g_ref[...] + be_ref[...]
    o_ref[...] = jnp.maximum(y, 0.0)


def _dense(x, w, bvec, gamma, beta):
    blk = 2048
    return pl.pallas_call(
        _dense_body,
        grid=(B // blk,),
        in_specs=[
            pl.BlockSpec((blk, D), lambda i: (i, 0)),
            pl.BlockSpec((K, D), lambda i: (0, 0)),
            pl.BlockSpec((1, K), lambda i: (0, 0)),
            pl.BlockSpec((1, K), lambda i: (0, 0)),
            pl.BlockSpec((1, K), lambda i: (0, 0)),
        ],
        out_specs=pl.BlockSpec((blk, K), lambda i: (i, 0)),
        out_shape=jax.ShapeDtypeStruct((B, K), jnp.float32),
    )(x, w, bvec, gamma, beta)


def kernel(keyword_ids, keyword_mask, table, W, b, gamma, beta):
    ids2 = keyword_ids.reshape(B // RPG, IPG).astype(jnp.int32)
    mask2 = keyword_mask.reshape(B // RPG, IPG)
    pooled = _pool_kernel(ids2, mask2, table.astype(jnp.bfloat16))
    return _dense(pooled.reshape(B, D), W, b.reshape(1, K),
                  gamma.reshape(1, K), beta.reshape(1, K))
